# Initial kernel scaffold; baseline (speedup 1.0000x reference)
import jax
import jax.numpy as jnp
from jax.experimental import pallas as pl


def kernel(B_z, G_z, x_b_batch, x_g_batch, W, b):
    raise NotImplementedError("write your pallas kernel here")


# speedup vs baseline: 4.3391x; 4.3391x over previous
import jax
import jax.numpy as jnp
from jax import lax
from jax.experimental import pallas as pl
from jax.experimental.pallas import tpu as pltpu
from jax.experimental.pallas import tpu_sc as plsc

_NB = 100000
_C = 128
_S = 256
_R = 80
_NCH = _NB // _R
_NW = 32
_CHUNKS_BASE = _NCH // _NW
_CHUNKS_EXTRA = _NCH % _NW

_BK = 2000
_NBLK = _NB // _BK


def _sc_segment_sums(b_hbm, g_hbm, ib_hbm, ig_hbm, z_acc_hbm,
                     accb_out, accg_out,
                     b_buf, g_buf, ib_buf, ig_buf, accb_sh, accg_sh):
    cid = lax.axis_index("c")
    sid = lax.axis_index("s")
    wid = cid * 16 + sid

    @pl.when(sid == 0)
    def _init():
        pltpu.sync_copy(z_acc_hbm, accb_sh)
        pltpu.sync_copy(z_acc_hbm, accg_sh)

    plsc.subcore_barrier()

    nch = jnp.where(wid < _CHUNKS_EXTRA, _CHUNKS_BASE + 1, _CHUNKS_BASE)

    def chunk_body(i, carry):
        base = (wid + i * _NW) * _R
        pltpu.sync_copy(b_hbm.at[pl.ds(base, _R)], b_buf)
        pltpu.sync_copy(ib_hbm.at[pl.ds(base, _R)], ib_buf)
        pltpu.sync_copy(b_buf, accb_sh.at[ib_buf], add=True)
        pltpu.sync_copy(g_hbm.at[pl.ds(base, _R)], g_buf)
        pltpu.sync_copy(ig_hbm.at[pl.ds(base, _R)], ig_buf)
        pltpu.sync_copy(g_buf, accg_sh.at[ig_buf], add=True)
        return carry

    lax.fori_loop(0, nch, chunk_body, 0)
    plsc.subcore_barrier()

    @pl.when(sid == 0)
    def _flush():
        pltpu.sync_copy(accb_sh, accb_out.at[cid])
        pltpu.sync_copy(accg_sh, accg_out.at[cid])


def _hist16(ids_row):
    hi = ids_row // 16
    lo = ids_row % 16
    k = lax.broadcasted_iota(jnp.int32, (16, _BK), 0)
    oh_hi = (k == hi).astype(jnp.float32)
    oh_lo = (k == lo).astype(jnp.float32)
    return lax.dot_general(oh_hi, oh_lo, (((1,), (1,)), ((), ())),
                           preferred_element_type=jnp.float32)


def _expand_counts(c16):
    g = lax.broadcasted_iota(jnp.int32, (_S, 16), 0)
    k = lax.broadcasted_iota(jnp.int32, (_S, 16), 1)
    sel_hi = (g // 16 == k).astype(jnp.float32)
    sel_lo = (g % 16 == k).astype(jnp.float32)
    rows = jnp.dot(sel_hi, c16, preferred_element_type=jnp.float32)
    return jnp.sum(rows * sel_lo, axis=1, keepdims=True)


def _combine_kernel(idb_ref, idg_ref, accb_ref, accg_ref, w_ref, bias_ref,
                    out_ref, cb16, cg16):
    i = pl.program_id(0)

    @pl.when(i == 0)
    def _zero():
        cb16[:, :] = jnp.zeros((16, 16), jnp.float32)
        cg16[:, :] = jnp.zeros((16, 16), jnp.float32)

    cb16[:, :] += _hist16(idb_ref[0])
    cg16[:, :] += _hist16(idg_ref[0])

    @pl.when(i == _NBLK - 1)
    def _final():
        cb = _expand_counts(cb16[:, :])
        cg = _expand_counts(cg16[:, :])
        sb = accb_ref[0] + accb_ref[1]
        sg = accg_ref[0] + accg_ref[1]
        hb = sb / jnp.maximum(cb, 1.0)
        hg = sg / jnp.maximum(cg, 1.0)
        w1 = w_ref[0:_C, :]
        w2 = w_ref[_C:2 * _C, :]
        out_ref[:, :] = (
            jnp.dot(hb, w1, preferred_element_type=jnp.float32)
            + jnp.dot(hg, w2, preferred_element_type=jnp.float32)
            + bias_ref[0, 0]
        )


def kernel(B_z, G_z, x_b_batch, x_g_batch, W, b):
    ib = x_b_batch.astype(jnp.int32)
    ig = x_g_batch.astype(jnp.int32)
    z_acc = jnp.zeros((_S, _C), jnp.float32)

    mesh = plsc.VectorSubcoreMesh(core_axis_name="c", subcore_axis_name="s")
    sc = pl.kernel(
        _sc_segment_sums,
        out_type=(
            jax.ShapeDtypeStruct((2, _S, _C), jnp.float32),
            jax.ShapeDtypeStruct((2, _S, _C), jnp.float32),
        ),
        mesh=mesh,
        scratch_types=[
            pltpu.VMEM((_R, _C), jnp.float32),
            pltpu.VMEM((_R, _C), jnp.float32),
            pltpu.VMEM((_R,), jnp.int32),
            pltpu.VMEM((_R,), jnp.int32),
            pltpu.VMEM_SHARED((_S, _C), jnp.float32),
            pltpu.VMEM_SHARED((_S, _C), jnp.float32),
        ],
    )
    accb, accg = sc(B_z, G_z, ib, ig, z_acc)

    out = pl.pallas_call(
        _combine_kernel,
        grid=(_NBLK,),
        in_specs=[
            pl.BlockSpec((1, 1, _BK), lambda i: (i, 0, 0)),
            pl.BlockSpec((1, 1, _BK), lambda i: (i, 0, 0)),
            pl.BlockSpec((2, _S, _C), lambda i: (0, 0, 0)),
            pl.BlockSpec((2, _S, _C), lambda i: (0, 0, 0)),
            pl.BlockSpec((2 * _C, 1), lambda i: (0, 0)),
            pl.BlockSpec((1, 1), lambda i: (0, 0)),
        ],
        out_specs=pl.BlockSpec((_S, 1), lambda i: (0, 0)),
        out_shape=jax.ShapeDtypeStruct((_S, 1), jnp.float32),
        scratch_shapes=[
            pltpu.VMEM((16, 16), jnp.float32),
            pltpu.VMEM((16, 16), jnp.float32),
        ],
    )(ib.reshape(_NBLK, 1, _BK), ig.reshape(_NBLK, 1, _BK),
      accb, accg, W, b.reshape(1, 1))
    return out


# speedup vs baseline: 7.6964x; 1.7737x over previous
import jax
import jax.numpy as jnp
from jax import lax
from jax.experimental import pallas as pl
from jax.experimental.pallas import tpu as pltpu
from jax.experimental.pallas import tpu_sc as plsc

_NB = 100000
_C = 128
_S = 256
_R = 80
_NCH = _NB // _R
_NW = 32
_CHUNKS_BASE = _NCH // _NW
_CHUNKS_EXTRA = _NCH % _NW

_BK = 2000
_NBLK = _NB // _BK


def _sc_segment_sums(b_hbm, g_hbm, ib_hbm, ig_hbm, z_acc_hbm,
                     accb_out, accg_out,
                     bb0, bb1, gb0, gb1, ib0, ib1, ig0, ig1,
                     accb_sh, accg_sh, ld0, ld1, st0, st1):
    cid = lax.axis_index("c")
    sid = lax.axis_index("s")
    wid = cid * 16 + sid

    bufs = ((bb0, gb0, ib0, ig0, ld0, st0), (bb1, gb1, ib1, ig1, ld1, st1))

    def start_load(i, s):
        bb, gb, ibuf, igbuf, ld, _ = bufs[s]
        base = (wid + i * _NW) * _R
        pltpu.async_copy(b_hbm.at[pl.ds(base, _R)], bb, ld)
        pltpu.async_copy(ib_hbm.at[pl.ds(base, _R)], ibuf, ld)
        pltpu.async_copy(g_hbm.at[pl.ds(base, _R)], gb, ld)
        pltpu.async_copy(ig_hbm.at[pl.ds(base, _R)], igbuf, ld)

    def wait_load(s):
        bb, gb, ibuf, igbuf, ld, _ = bufs[s]
        pltpu.make_async_copy(b_hbm.at[pl.ds(0, _R)], bb, ld).wait()
        pltpu.make_async_copy(ib_hbm.at[pl.ds(0, _R)], ibuf, ld).wait()
        pltpu.make_async_copy(g_hbm.at[pl.ds(0, _R)], gb, ld).wait()
        pltpu.make_async_copy(ig_hbm.at[pl.ds(0, _R)], igbuf, ld).wait()

    def start_scatter(s):
        bb, gb, ibuf, igbuf, _, st = bufs[s]
        pltpu.async_copy(bb, accb_sh.at[ibuf], st, add=True)
        pltpu.async_copy(gb, accg_sh.at[igbuf], st, add=True)

    def wait_scatter(s):
        bb, gb, ibuf, igbuf, _, st = bufs[s]
        pltpu.make_async_copy(bb, accb_sh.at[ibuf], st).wait()
        pltpu.make_async_copy(gb, accg_sh.at[igbuf], st).wait()

    nch = jnp.where(wid < _CHUNKS_EXTRA, _CHUNKS_BASE + 1, _CHUNKS_BASE)

    start_load(0, 0)

    @pl.when(sid == 0)
    def _init():
        pltpu.sync_copy(z_acc_hbm, accb_sh)
        pltpu.sync_copy(z_acc_hbm, accg_sh)

    plsc.subcore_barrier()

    def outer(k, carry):
        for s in (0, 1):
            i = 2 * k + s

            @pl.when(i < nch)
            def _step():
                wait_load(s)
                start_scatter(s)

                @pl.when(i + 1 < nch)
                def _prefetch():
                    @pl.when(i >= 1)
                    def _drain():
                        wait_scatter(1 - s)

                    start_load(i + 1, 1 - s)

        return carry

    lax.fori_loop(0, (_CHUNKS_BASE + 2) // 2, outer, 0)
    wait_scatter(0)
    wait_scatter(1)
    plsc.subcore_barrier()

    @pl.when(sid == 0)
    def _flush():
        pltpu.sync_copy(accb_sh, accb_out.at[cid])
        pltpu.sync_copy(accg_sh, accg_out.at[cid])


def _hist16(ids_row):
    hi = ids_row // 16
    lo = ids_row % 16
    k = lax.broadcasted_iota(jnp.int32, (16, _BK), 0)
    oh_hi = (k == hi).astype(jnp.float32)
    oh_lo = (k == lo).astype(jnp.float32)
    return lax.dot_general(oh_hi, oh_lo, (((1,), (1,)), ((), ())),
                           preferred_element_type=jnp.float32)


def _expand_counts(c16):
    g = lax.broadcasted_iota(jnp.int32, (_S, 16), 0)
    k = lax.broadcasted_iota(jnp.int32, (_S, 16), 1)
    sel_hi = (g // 16 == k).astype(jnp.float32)
    sel_lo = (g % 16 == k).astype(jnp.float32)
    rows = jnp.dot(sel_hi, c16, preferred_element_type=jnp.float32)
    return jnp.sum(rows * sel_lo, axis=1, keepdims=True)


def _combine_kernel(idb_ref, idg_ref, accb_ref, accg_ref, w_ref, bias_ref,
                    out_ref, cb16, cg16):
    i = pl.program_id(0)

    @pl.when(i == 0)
    def _zero():
        cb16[:, :] = jnp.zeros((16, 16), jnp.float32)
        cg16[:, :] = jnp.zeros((16, 16), jnp.float32)

    cb16[:, :] += _hist16(idb_ref[0])
    cg16[:, :] += _hist16(idg_ref[0])

    @pl.when(i == _NBLK - 1)
    def _final():
        cb = _expand_counts(cb16[:, :])
        cg = _expand_counts(cg16[:, :])
        sb = accb_ref[0] + accb_ref[1]
        sg = accg_ref[0] + accg_ref[1]
        hb = sb / jnp.maximum(cb, 1.0)
        hg = sg / jnp.maximum(cg, 1.0)
        w1 = w_ref[0:_C, :]
        w2 = w_ref[_C:2 * _C, :]
        out_ref[:, :] = (
            jnp.dot(hb, w1, preferred_element_type=jnp.float32)
            + jnp.dot(hg, w2, preferred_element_type=jnp.float32)
            + bias_ref[0, 0]
        )


def kernel(B_z, G_z, x_b_batch, x_g_batch, W, b):
    ib = x_b_batch.astype(jnp.int32)
    ig = x_g_batch.astype(jnp.int32)
    z_acc = jnp.zeros((_S, _C), jnp.float32)

    mesh = plsc.VectorSubcoreMesh(core_axis_name="c", subcore_axis_name="s")
    sc = pl.kernel(
        _sc_segment_sums,
        out_type=(
            jax.ShapeDtypeStruct((2, _S, _C), jnp.float32),
            jax.ShapeDtypeStruct((2, _S, _C), jnp.float32),
        ),
        mesh=mesh,
        scratch_types=[
            pltpu.VMEM((_R, _C), jnp.float32),
            pltpu.VMEM((_R, _C), jnp.float32),
            pltpu.VMEM((_R, _C), jnp.float32),
            pltpu.VMEM((_R, _C), jnp.float32),
            pltpu.VMEM((_R,), jnp.int32),
            pltpu.VMEM((_R,), jnp.int32),
            pltpu.VMEM((_R,), jnp.int32),
            pltpu.VMEM((_R,), jnp.int32),
            pltpu.VMEM_SHARED((_S, _C), jnp.float32),
            pltpu.VMEM_SHARED((_S, _C), jnp.float32),
            pltpu.SemaphoreType.DMA,
            pltpu.SemaphoreType.DMA,
            pltpu.SemaphoreType.DMA,
            pltpu.SemaphoreType.DMA,
        ],
    )
    accb, accg = sc(B_z, G_z, ib, ig, z_acc)

    out = pl.pallas_call(
        _combine_kernel,
        grid=(_NBLK,),
        in_specs=[
            pl.BlockSpec((1, 1, _BK), lambda i: (i, 0, 0)),
            pl.BlockSpec((1, 1, _BK), lambda i: (i, 0, 0)),
            pl.BlockSpec((2, _S, _C), lambda i: (0, 0, 0)),
            pl.BlockSpec((2, _S, _C), lambda i: (0, 0, 0)),
            pl.BlockSpec((2 * _C, 1), lambda i: (0, 0)),
            pl.BlockSpec((1, 1), lambda i: (0, 0)),
        ],
        out_specs=pl.BlockSpec((_S, 1), lambda i: (0, 0)),
        out_shape=jax.ShapeDtypeStruct((_S, 1), jnp.float32),
        scratch_shapes=[
            pltpu.VMEM((16, 16), jnp.float32),
            pltpu.VMEM((16, 16), jnp.float32),
        ],
    )(ib.reshape(_NBLK, 1, _BK), ig.reshape(_NBLK, 1, _BK),
      accb, accg, W, b.reshape(1, 1))
    return out


# speedup vs baseline: 9.7602x; 1.2682x over previous
import jax
import jax.numpy as jnp
from jax import lax
from jax.experimental import pallas as pl
from jax.experimental.pallas import tpu as pltpu
from jax.experimental.pallas import tpu_sc as plsc

_NB = 100000
_C = 128
_S = 256
_R = 80
_NCH = _NB // _R
_NW = 32
_CHUNKS_BASE = _NCH // _NW
_CHUNKS_EXTRA = _NCH % _NW

_BK = 2000
_NBLK = _NB // _BK


def _sc_segment_sums(b_hbm, g_hbm, ib_hbm, ig_hbm, z_acc_hbm,
                     accb_out, accg_out,
                     bb0, bb1, gb0, gb1, ib0, ib1, ig0, ig1,
                     accb_sh, accg_sh, ld0, ld1, st0, st1):
    cid = lax.axis_index("c")
    sid = lax.axis_index("s")
    wid = cid * 16 + sid

    bufs = ((bb0, gb0, ib0, ig0, ld0, st0), (bb1, gb1, ib1, ig1, ld1, st1))

    def start_load(i, s):
        bb, gb, ibuf, igbuf, ld, _ = bufs[s]
        base = (wid + i * _NW) * _R
        pltpu.async_copy(b_hbm.at[pl.ds(base, _R)], bb, ld)
        pltpu.async_copy(ib_hbm.at[pl.ds(base, _R)], ibuf, ld)
        pltpu.async_copy(g_hbm.at[pl.ds(base, _R)], gb, ld)
        pltpu.async_copy(ig_hbm.at[pl.ds(base, _R)], igbuf, ld)

    def wait_load(s):
        bb, gb, ibuf, igbuf, ld, _ = bufs[s]
        pltpu.make_async_copy(b_hbm.at[pl.ds(0, _R)], bb, ld).wait()
        pltpu.make_async_copy(ib_hbm.at[pl.ds(0, _R)], ibuf, ld).wait()
        pltpu.make_async_copy(g_hbm.at[pl.ds(0, _R)], gb, ld).wait()
        pltpu.make_async_copy(ig_hbm.at[pl.ds(0, _R)], igbuf, ld).wait()

    def start_scatter(s):
        bb, gb, ibuf, igbuf, _, st = bufs[s]
        pltpu.async_copy(bb, accb_sh.at[ibuf], st, add=True)
        pltpu.async_copy(gb, accg_sh.at[igbuf], st, add=True)

    def wait_scatter(s):
        bb, gb, ibuf, igbuf, _, st = bufs[s]
        pltpu.make_async_copy(bb, accb_sh.at[ibuf], st).wait()
        pltpu.make_async_copy(gb, accg_sh.at[igbuf], st).wait()

    nch = jnp.where(wid < _CHUNKS_EXTRA, _CHUNKS_BASE + 1, _CHUNKS_BASE)

    start_load(0, 0)

    @pl.when(sid == 0)
    def _init():
        pltpu.sync_copy(z_acc_hbm, accb_sh)
        pltpu.sync_copy(z_acc_hbm, accg_sh)

    plsc.subcore_barrier()

    def outer(k, carry):
        for s in (0, 1):
            i = 2 * k + s

            @pl.when(i < nch)
            def _step():
                wait_load(s)
                start_scatter(s)

                @pl.when(i + 1 < nch)
                def _prefetch():
                    @pl.when(i >= 1)
                    def _drain():
                        wait_scatter(1 - s)

                    start_load(i + 1, 1 - s)

        return carry

    lax.fori_loop(0, (_CHUNKS_BASE + 2) // 2, outer, 0)
    wait_scatter(0)
    wait_scatter(1)
    plsc.subcore_barrier()

    @pl.when(sid == 0)
    def _flush():
        pltpu.sync_copy(accb_sh, accb_out.at[cid])
        pltpu.sync_copy(accg_sh, accg_out.at[cid])


def _hist16(ids_row):
    hi = ids_row // 16
    lo = ids_row % 16
    k = lax.broadcasted_iota(jnp.int32, (16, _BK), 0)
    oh_hi = (k == hi).astype(jnp.float32)
    oh_lo = (k == lo).astype(jnp.float32)
    return lax.dot_general(oh_hi, oh_lo, (((1,), (1,)), ((), ())),
                           preferred_element_type=jnp.float32)


def _expand_counts(c16):
    g = lax.broadcasted_iota(jnp.int32, (_S, 16), 0)
    k = lax.broadcasted_iota(jnp.int32, (_S, 16), 1)
    sel_hi = (g // 16 == k).astype(jnp.float32)
    sel_lo = (g % 16 == k).astype(jnp.float32)
    rows = jnp.dot(sel_hi, c16, preferred_element_type=jnp.float32)
    return jnp.sum(rows * sel_lo, axis=1, keepdims=True)


def _counts_kernel(idb_ref, idg_ref, cb_out, cg_out, cb16, cg16):
    i = pl.program_id(0)

    @pl.when(i == 0)
    def _zero():
        cb16[:, :] = jnp.zeros((16, 16), jnp.float32)
        cg16[:, :] = jnp.zeros((16, 16), jnp.float32)

    cb16[:, :] += _hist16(idb_ref[0])
    cg16[:, :] += _hist16(idg_ref[0])

    @pl.when(i == _NBLK - 1)
    def _final():
        cb_out[:, :] = _expand_counts(cb16[:, :])
        cg_out[:, :] = _expand_counts(cg16[:, :])


def _combine_kernel(accb_ref, accg_ref, cb_ref, cg_ref, w_ref, bias_ref,
                    out_ref):
    sb = accb_ref[0] + accb_ref[1]
    sg = accg_ref[0] + accg_ref[1]
    hb = sb / jnp.maximum(cb_ref[:, :], 1.0)
    hg = sg / jnp.maximum(cg_ref[:, :], 1.0)
    w1 = w_ref[0:_C, :]
    w2 = w_ref[_C:2 * _C, :]
    out_ref[:, :] = (
        jnp.dot(hb, w1, preferred_element_type=jnp.float32)
        + jnp.dot(hg, w2, preferred_element_type=jnp.float32)
        + bias_ref[0, 0]
    )


def kernel(B_z, G_z, x_b_batch, x_g_batch, W, b):
    ib = x_b_batch.astype(jnp.int32)
    ig = x_g_batch.astype(jnp.int32)
    z_acc = jnp.zeros((_S, _C), jnp.float32)

    mesh = plsc.VectorSubcoreMesh(core_axis_name="c", subcore_axis_name="s")
    sc = pl.kernel(
        _sc_segment_sums,
        out_type=(
            jax.ShapeDtypeStruct((2, _S, _C), jnp.float32),
            jax.ShapeDtypeStruct((2, _S, _C), jnp.float32),
        ),
        mesh=mesh,
        scratch_types=[
            pltpu.VMEM((_R, _C), jnp.float32),
            pltpu.VMEM((_R, _C), jnp.float32),
            pltpu.VMEM((_R, _C), jnp.float32),
            pltpu.VMEM((_R, _C), jnp.float32),
            pltpu.VMEM((_R,), jnp.int32),
            pltpu.VMEM((_R,), jnp.int32),
            pltpu.VMEM((_R,), jnp.int32),
            pltpu.VMEM((_R,), jnp.int32),
            pltpu.VMEM_SHARED((_S, _C), jnp.float32),
            pltpu.VMEM_SHARED((_S, _C), jnp.float32),
            pltpu.SemaphoreType.DMA,
            pltpu.SemaphoreType.DMA,
            pltpu.SemaphoreType.DMA,
            pltpu.SemaphoreType.DMA,
        ],
    )
    accb, accg = sc(B_z, G_z, ib, ig, z_acc)

    cb, cg = pl.pallas_call(
        _counts_kernel,
        grid=(_NBLK,),
        in_specs=[
            pl.BlockSpec((1, 1, _BK), lambda i: (i, 0, 0)),
            pl.BlockSpec((1, 1, _BK), lambda i: (i, 0, 0)),
        ],
        out_specs=[
            pl.BlockSpec((_S, 1), lambda i: (0, 0)),
            pl.BlockSpec((_S, 1), lambda i: (0, 0)),
        ],
        out_shape=[
            jax.ShapeDtypeStruct((_S, 1), jnp.float32),
            jax.ShapeDtypeStruct((_S, 1), jnp.float32),
        ],
        scratch_shapes=[
            pltpu.VMEM((16, 16), jnp.float32),
            pltpu.VMEM((16, 16), jnp.float32),
        ],
    )(ib.reshape(_NBLK, 1, _BK), ig.reshape(_NBLK, 1, _BK))

    out = pl.pallas_call(
        _combine_kernel,
        out_shape=jax.ShapeDtypeStruct((_S, 1), jnp.float32),
    )(accb, accg, cb, cg, W, b.reshape(1, 1))
    return out


# speedup vs baseline: 9.7683x; 1.0008x over previous
import jax
import jax.numpy as jnp
from jax import lax
from jax.experimental import pallas as pl
from jax.experimental.pallas import tpu as pltpu
from jax.experimental.pallas import tpu_sc as plsc

_NB = 100000
_C = 128
_S = 256
_R = 80
_NCH = _NB // _R
_NW = 32
_CHUNKS_BASE = _NCH // _NW
_CHUNKS_EXTRA = _NCH % _NW

_BK = 25000
_NBLK = _NB // _BK


def _sc_segment_sums(b_hbm, g_hbm, ib_hbm, ig_hbm, z_acc_hbm,
                     accb_out, accg_out,
                     bb0, bb1, gb0, gb1, ib0, ib1, ig0, ig1,
                     accb_sh, accg_sh, ld0, ld1, st0, st1):
    cid = lax.axis_index("c")
    sid = lax.axis_index("s")
    wid = cid * 16 + sid

    bufs = ((bb0, gb0, ib0, ig0, ld0, st0), (bb1, gb1, ib1, ig1, ld1, st1))

    def start_load(i, s):
        bb, gb, ibuf, igbuf, ld, _ = bufs[s]
        base = (wid + i * _NW) * _R
        pltpu.async_copy(b_hbm.at[pl.ds(base, _R)], bb, ld)
        pltpu.async_copy(ib_hbm.at[pl.ds(base, _R)], ibuf, ld)
        pltpu.async_copy(g_hbm.at[pl.ds(base, _R)], gb, ld)
        pltpu.async_copy(ig_hbm.at[pl.ds(base, _R)], igbuf, ld)

    def wait_load(s):
        bb, gb, ibuf, igbuf, ld, _ = bufs[s]
        pltpu.make_async_copy(b_hbm.at[pl.ds(0, _R)], bb, ld).wait()
        pltpu.make_async_copy(ib_hbm.at[pl.ds(0, _R)], ibuf, ld).wait()
        pltpu.make_async_copy(g_hbm.at[pl.ds(0, _R)], gb, ld).wait()
        pltpu.make_async_copy(ig_hbm.at[pl.ds(0, _R)], igbuf, ld).wait()

    def start_scatter(s):
        bb, gb, ibuf, igbuf, _, st = bufs[s]
        pltpu.async_copy(bb, accb_sh.at[ibuf], st, add=True)
        pltpu.async_copy(gb, accg_sh.at[igbuf], st, add=True)

    def wait_scatter(s):
        bb, gb, ibuf, igbuf, _, st = bufs[s]
        pltpu.make_async_copy(bb, accb_sh.at[ibuf], st).wait()
        pltpu.make_async_copy(gb, accg_sh.at[igbuf], st).wait()

    nch = jnp.where(wid < _CHUNKS_EXTRA, _CHUNKS_BASE + 1, _CHUNKS_BASE)

    start_load(0, 0)

    @pl.when(sid == 0)
    def _init():
        pltpu.sync_copy(z_acc_hbm, accb_sh)
        pltpu.sync_copy(z_acc_hbm, accg_sh)

    plsc.subcore_barrier()

    def outer(k, carry):
        for s in (0, 1):
            i = 2 * k + s

            @pl.when(i < nch)
            def _step():
                wait_load(s)
                start_scatter(s)

                @pl.when(i + 1 < nch)
                def _prefetch():
                    @pl.when(i >= 1)
                    def _drain():
                        wait_scatter(1 - s)

                    start_load(i + 1, 1 - s)

        return carry

    lax.fori_loop(0, (_CHUNKS_BASE + 2) // 2, outer, 0)
    wait_scatter(0)
    wait_scatter(1)
    plsc.subcore_barrier()

    @pl.when(sid == 0)
    def _flush():
        pltpu.sync_copy(accb_sh, accb_out.at[cid])
        pltpu.sync_copy(accg_sh, accg_out.at[cid])


def _hist16(ids_row):
    hi = lax.shift_right_logical(ids_row, 4)
    lo = ids_row & 15
    k = lax.broadcasted_iota(jnp.int32, (16, _BK), 0)
    oh_hi = (k == hi).astype(jnp.float32)
    oh_lo = (k == lo).astype(jnp.float32)
    return lax.dot_general(oh_hi, oh_lo, (((1,), (1,)), ((), ())),
                           preferred_element_type=jnp.float32)


def _expand_counts(c16):
    g = lax.broadcasted_iota(jnp.int32, (_S, 16), 0)
    k = lax.broadcasted_iota(jnp.int32, (_S, 16), 1)
    sel_hi = (g // 16 == k).astype(jnp.float32)
    sel_lo = (g % 16 == k).astype(jnp.float32)
    rows = jnp.dot(sel_hi, c16, preferred_element_type=jnp.float32)
    return jnp.sum(rows * sel_lo, axis=1, keepdims=True)


def _counts_kernel(idb_ref, idg_ref, cb_out, cg_out, cb16, cg16):
    i = pl.program_id(0)

    @pl.when(i == 0)
    def _zero():
        cb16[:, :] = jnp.zeros((16, 16), jnp.float32)
        cg16[:, :] = jnp.zeros((16, 16), jnp.float32)

    cb16[:, :] += _hist16(idb_ref[0])
    cg16[:, :] += _hist16(idg_ref[0])

    @pl.when(i == _NBLK - 1)
    def _final():
        cb_out[:, :] = _expand_counts(cb16[:, :])
        cg_out[:, :] = _expand_counts(cg16[:, :])


def _combine_kernel(accb_ref, accg_ref, cb_ref, cg_ref, w_ref, bias_ref,
                    out_ref):
    sb = accb_ref[0] + accb_ref[1]
    sg = accg_ref[0] + accg_ref[1]
    hb = sb / jnp.maximum(cb_ref[:, :], 1.0)
    hg = sg / jnp.maximum(cg_ref[:, :], 1.0)
    w1 = w_ref[0:_C, :]
    w2 = w_ref[_C:2 * _C, :]
    out_ref[:, :] = (
        jnp.dot(hb, w1, preferred_element_type=jnp.float32)
        + jnp.dot(hg, w2, preferred_element_type=jnp.float32)
        + bias_ref[0, 0]
    )


def kernel(B_z, G_z, x_b_batch, x_g_batch, W, b):
    ib = x_b_batch.astype(jnp.int32)
    ig = x_g_batch.astype(jnp.int32)
    z_acc = jnp.zeros((_S, _C), jnp.float32)

    mesh = plsc.VectorSubcoreMesh(core_axis_name="c", subcore_axis_name="s")
    sc = pl.kernel(
        _sc_segment_sums,
        out_type=(
            jax.ShapeDtypeStruct((2, _S, _C), jnp.float32),
            jax.ShapeDtypeStruct((2, _S, _C), jnp.float32),
        ),
        mesh=mesh,
        scratch_types=[
            pltpu.VMEM((_R, _C), jnp.float32),
            pltpu.VMEM((_R, _C), jnp.float32),
            pltpu.VMEM((_R, _C), jnp.float32),
            pltpu.VMEM((_R, _C), jnp.float32),
            pltpu.VMEM((_R,), jnp.int32),
            pltpu.VMEM((_R,), jnp.int32),
            pltpu.VMEM((_R,), jnp.int32),
            pltpu.VMEM((_R,), jnp.int32),
            pltpu.VMEM_SHARED((_S, _C), jnp.float32),
            pltpu.VMEM_SHARED((_S, _C), jnp.float32),
            pltpu.SemaphoreType.DMA,
            pltpu.SemaphoreType.DMA,
            pltpu.SemaphoreType.DMA,
            pltpu.SemaphoreType.DMA,
        ],
    )
    accb, accg = sc(B_z, G_z, ib, ig, z_acc)

    cb, cg = pl.pallas_call(
        _counts_kernel,
        grid=(_NBLK,),
        in_specs=[
            pl.BlockSpec((1, 1, _BK), lambda i: (i, 0, 0)),
            pl.BlockSpec((1, 1, _BK), lambda i: (i, 0, 0)),
        ],
        out_specs=[
            pl.BlockSpec((_S, 1), lambda i: (0, 0)),
            pl.BlockSpec((_S, 1), lambda i: (0, 0)),
        ],
        out_shape=[
            jax.ShapeDtypeStruct((_S, 1), jnp.float32),
            jax.ShapeDtypeStruct((_S, 1), jnp.float32),
        ],
        scratch_shapes=[
            pltpu.VMEM((16, 16), jnp.float32),
            pltpu.VMEM((16, 16), jnp.float32),
        ],
    )(ib.reshape(_NBLK, 1, _BK), ig.reshape(_NBLK, 1, _BK))

    out = pl.pallas_call(
        _combine_kernel,
        out_shape=jax.ShapeDtypeStruct((_S, 1), jnp.float32),
    )(accb, accg, cb, cg, W, b.reshape(1, 1))
    return out


# speedup vs baseline: 12.3917x; 1.2686x over previous
import jax
import jax.numpy as jnp
from jax import lax
from jax.experimental import pallas as pl
from jax.experimental.pallas import tpu as pltpu
from jax.experimental.pallas import tpu_sc as plsc

_NB = 100000
_C = 128
_S = 256
_R = 80
_ROWS_SC = 64000
_NCH = _ROWS_SC // _R
_NW = 32
_CHUNKS_BASE = _NCH // _NW
_CHUNKS_EXTRA = _NCH % _NW

_BK = 25000
_NBLK = _NB // _BK

_TBK = 2000
_NT = (_NB - _ROWS_SC) // _TBK
_OFF = _ROWS_SC // _TBK


def _sc_segment_sums(b_hbm, g_hbm, ib_hbm, ig_hbm, z_acc_hbm,
                     accb_out, accg_out,
                     bb0, bb1, gb0, gb1, ib0, ib1, ig0, ig1,
                     accb_sh, accg_sh, ld0, ld1, st0, st1):
    cid = lax.axis_index("c")
    sid = lax.axis_index("s")
    wid = cid * 16 + sid

    bufs = ((bb0, gb0, ib0, ig0, ld0, st0), (bb1, gb1, ib1, ig1, ld1, st1))

    def start_load(i, s):
        bb, gb, ibuf, igbuf, ld, _ = bufs[s]
        base = (wid + i * _NW) * _R
        pltpu.async_copy(b_hbm.at[pl.ds(base, _R)], bb, ld)
        pltpu.async_copy(ib_hbm.at[pl.ds(base, _R)], ibuf, ld)
        pltpu.async_copy(g_hbm.at[pl.ds(base, _R)], gb, ld)
        pltpu.async_copy(ig_hbm.at[pl.ds(base, _R)], igbuf, ld)

    def wait_load(s):
        bb, gb, ibuf, igbuf, ld, _ = bufs[s]
        pltpu.make_async_copy(b_hbm.at[pl.ds(0, _R)], bb, ld).wait()
        pltpu.make_async_copy(ib_hbm.at[pl.ds(0, _R)], ibuf, ld).wait()
        pltpu.make_async_copy(g_hbm.at[pl.ds(0, _R)], gb, ld).wait()
        pltpu.make_async_copy(ig_hbm.at[pl.ds(0, _R)], igbuf, ld).wait()

    def start_scatter(s):
        bb, gb, ibuf, igbuf, _, st = bufs[s]
        pltpu.async_copy(bb, accb_sh.at[ibuf], st, add=True)
        pltpu.async_copy(gb, accg_sh.at[igbuf], st, add=True)

    def wait_scatter(s):
        bb, gb, ibuf, igbuf, _, st = bufs[s]
        pltpu.make_async_copy(bb, accb_sh.at[ibuf], st).wait()
        pltpu.make_async_copy(gb, accg_sh.at[igbuf], st).wait()

    nch = jnp.where(wid < _CHUNKS_EXTRA, _CHUNKS_BASE + 1, _CHUNKS_BASE)

    start_load(0, 0)

    @pl.when(sid == 0)
    def _init():
        pltpu.sync_copy(z_acc_hbm, accb_sh)
        pltpu.sync_copy(z_acc_hbm, accg_sh)

    plsc.subcore_barrier()

    def outer(k, carry):
        for s in (0, 1):
            i = 2 * k + s

            @pl.when(i < nch)
            def _step():
                wait_load(s)
                start_scatter(s)

                @pl.when(i + 1 < nch)
                def _prefetch():
                    @pl.when(i >= 1)
                    def _drain():
                        wait_scatter(1 - s)

                    start_load(i + 1, 1 - s)

        return carry

    lax.fori_loop(0, (_CHUNKS_BASE + 2) // 2, outer, 0)
    wait_scatter(0)
    wait_scatter(1)
    plsc.subcore_barrier()

    @pl.when(sid == 0)
    def _flush():
        pltpu.sync_copy(accb_sh, accb_out.at[cid])
        pltpu.sync_copy(accg_sh, accg_out.at[cid])


def _hist16(ids_row):
    hi = lax.shift_right_logical(ids_row, 4)
    lo = ids_row & 15
    k = lax.broadcasted_iota(jnp.int32, (16, _BK), 0)
    oh_hi = (k == hi).astype(jnp.float32)
    oh_lo = (k == lo).astype(jnp.float32)
    return lax.dot_general(oh_hi, oh_lo, (((1,), (1,)), ((), ())),
                           preferred_element_type=jnp.float32)


def _expand_counts(c16):
    g = lax.broadcasted_iota(jnp.int32, (_S, 16), 0)
    k = lax.broadcasted_iota(jnp.int32, (_S, 16), 1)
    sel_hi = (g // 16 == k).astype(jnp.float32)
    sel_lo = (g % 16 == k).astype(jnp.float32)
    rows = jnp.dot(sel_hi, c16, preferred_element_type=jnp.float32)
    return jnp.sum(rows * sel_lo, axis=1, keepdims=True)


def _counts_kernel(idb_ref, idg_ref, cb_out, cg_out, cb16, cg16):
    i = pl.program_id(0)

    @pl.when(i == 0)
    def _zero():
        cb16[:, :] = jnp.zeros((16, 16), jnp.float32)
        cg16[:, :] = jnp.zeros((16, 16), jnp.float32)

    cb16[:, :] += _hist16(idb_ref[0])
    cg16[:, :] += _hist16(idg_ref[0])

    @pl.when(i == _NBLK - 1)
    def _final():
        cb_out[:, :] = _expand_counts(cb16[:, :])
        cg_out[:, :] = _expand_counts(cg16[:, :])


def _tc_tail_segsum(idb_ref, idg_ref, bz_ref, gz_ref, tcb_out, tcg_out,
                    accb, accg):
    i = pl.program_id(0)

    @pl.when(i == 0)
    def _zero():
        accb[:, :] = jnp.zeros((_S, _C), jnp.float32)
        accg[:, :] = jnp.zeros((_S, _C), jnp.float32)

    g = lax.broadcasted_iota(jnp.int32, (_S, _TBK), 0)
    ohb = (g == idb_ref[0]).astype(jnp.float32)
    ohg = (g == idg_ref[0]).astype(jnp.float32)
    accb[:, :] += jnp.dot(ohb, bz_ref[:, :], preferred_element_type=jnp.float32)
    accg[:, :] += jnp.dot(ohg, gz_ref[:, :], preferred_element_type=jnp.float32)

    @pl.when(i == _NT - 1)
    def _final():
        tcb_out[:, :] = accb[:, :]
        tcg_out[:, :] = accg[:, :]


def _combine_kernel(accb_ref, accg_ref, tcb_ref, tcg_ref, cb_ref, cg_ref,
                    w_ref, bias_ref, out_ref):
    sb = accb_ref[0] + accb_ref[1] + tcb_ref[:, :]
    sg = accg_ref[0] + accg_ref[1] + tcg_ref[:, :]
    hb = sb / jnp.maximum(cb_ref[:, :], 1.0)
    hg = sg / jnp.maximum(cg_ref[:, :], 1.0)
    w1 = w_ref[0:_C, :]
    w2 = w_ref[_C:2 * _C, :]
    out_ref[:, :] = (
        jnp.dot(hb, w1, preferred_element_type=jnp.float32)
        + jnp.dot(hg, w2, preferred_element_type=jnp.float32)
        + bias_ref[0, 0]
    )


def kernel(B_z, G_z, x_b_batch, x_g_batch, W, b):
    ib = x_b_batch.astype(jnp.int32)
    ig = x_g_batch.astype(jnp.int32)
    z_acc = jnp.zeros((_S, _C), jnp.float32)

    mesh = plsc.VectorSubcoreMesh(core_axis_name="c", subcore_axis_name="s")
    sc = pl.kernel(
        _sc_segment_sums,
        out_type=(
            jax.ShapeDtypeStruct((2, _S, _C), jnp.float32),
            jax.ShapeDtypeStruct((2, _S, _C), jnp.float32),
        ),
        mesh=mesh,
        scratch_types=[
            pltpu.VMEM((_R, _C), jnp.float32),
            pltpu.VMEM((_R, _C), jnp.float32),
            pltpu.VMEM((_R, _C), jnp.float32),
            pltpu.VMEM((_R, _C), jnp.float32),
            pltpu.VMEM((_R,), jnp.int32),
            pltpu.VMEM((_R,), jnp.int32),
            pltpu.VMEM((_R,), jnp.int32),
            pltpu.VMEM((_R,), jnp.int32),
            pltpu.VMEM_SHARED((_S, _C), jnp.float32),
            pltpu.VMEM_SHARED((_S, _C), jnp.float32),
            pltpu.SemaphoreType.DMA,
            pltpu.SemaphoreType.DMA,
            pltpu.SemaphoreType.DMA,
            pltpu.SemaphoreType.DMA,
        ],
    )
    accb, accg = sc(B_z, G_z, ib, ig, z_acc)

    cb, cg = pl.pallas_call(
        _counts_kernel,
        grid=(_NBLK,),
        in_specs=[
            pl.BlockSpec((1, 1, _BK), lambda i: (i, 0, 0)),
            pl.BlockSpec((1, 1, _BK), lambda i: (i, 0, 0)),
        ],
        out_specs=[
            pl.BlockSpec((_S, 1), lambda i: (0, 0)),
            pl.BlockSpec((_S, 1), lambda i: (0, 0)),
        ],
        out_shape=[
            jax.ShapeDtypeStruct((_S, 1), jnp.float32),
            jax.ShapeDtypeStruct((_S, 1), jnp.float32),
        ],
        scratch_shapes=[
            pltpu.VMEM((16, 16), jnp.float32),
            pltpu.VMEM((16, 16), jnp.float32),
        ],
    )(ib.reshape(_NBLK, 1, _BK), ig.reshape(_NBLK, 1, _BK))

    tcb, tcg = pl.pallas_call(
        _tc_tail_segsum,
        grid=(_NT,),
        in_specs=[
            pl.BlockSpec((1, 1, _TBK), lambda i: (i + _OFF, 0, 0)),
            pl.BlockSpec((1, 1, _TBK), lambda i: (i + _OFF, 0, 0)),
            pl.BlockSpec((_TBK, _C), lambda i: (i + _OFF, 0)),
            pl.BlockSpec((_TBK, _C), lambda i: (i + _OFF, 0)),
        ],
        out_specs=[
            pl.BlockSpec((_S, _C), lambda i: (0, 0)),
            pl.BlockSpec((_S, _C), lambda i: (0, 0)),
        ],
        out_shape=[
            jax.ShapeDtypeStruct((_S, _C), jnp.float32),
            jax.ShapeDtypeStruct((_S, _C), jnp.float32),
        ],
        scratch_shapes=[
            pltpu.VMEM((_S, _C), jnp.float32),
            pltpu.VMEM((_S, _C), jnp.float32),
        ],
    )(ib.reshape(_NB // _TBK, 1, _TBK), ig.reshape(_NB // _TBK, 1, _TBK),
      B_z, G_z)

    out = pl.pallas_call(
        _combine_kernel,
        out_shape=jax.ShapeDtypeStruct((_S, 1), jnp.float32),
    )(accb, accg, tcb, tcg, cb, cg, W, b.reshape(1, 1))
    return out


# speedup vs baseline: 13.1313x; 1.0597x over previous
import jax
import jax.numpy as jnp
from jax import lax
from jax.experimental import pallas as pl
from jax.experimental.pallas import tpu as pltpu
from jax.experimental.pallas import tpu_sc as plsc

_NB = 100000
_C = 128
_S = 256
_R = 80
_ROWS_SC = 56000
_NCH = _ROWS_SC // _R
_NW = 32
_CHUNKS_BASE = _NCH // _NW
_CHUNKS_EXTRA = _NCH % _NW

_BK = 25000
_NBLK = _NB // _BK

_TBK = 2000
_NT = (_NB - _ROWS_SC) // _TBK
_OFF = _ROWS_SC // _TBK


def _sc_segment_sums(b_hbm, g_hbm, ib_hbm, ig_hbm, z_acc_hbm,
                     accb_out, accg_out,
                     bb0, bb1, gb0, gb1, ib0, ib1, ig0, ig1,
                     accb_sh, accg_sh, ld0, ld1, st0, st1):
    cid = lax.axis_index("c")
    sid = lax.axis_index("s")
    wid = cid * 16 + sid

    bufs = ((bb0, gb0, ib0, ig0, ld0, st0), (bb1, gb1, ib1, ig1, ld1, st1))

    def start_load(i, s):
        bb, gb, ibuf, igbuf, ld, _ = bufs[s]
        base = (wid + i * _NW) * _R
        pltpu.async_copy(b_hbm.at[pl.ds(base, _R)], bb, ld)
        pltpu.async_copy(ib_hbm.at[pl.ds(base, _R)], ibuf, ld)
        pltpu.async_copy(g_hbm.at[pl.ds(base, _R)], gb, ld)
        pltpu.async_copy(ig_hbm.at[pl.ds(base, _R)], igbuf, ld)

    def wait_load(s):
        bb, gb, ibuf, igbuf, ld, _ = bufs[s]
        pltpu.make_async_copy(b_hbm.at[pl.ds(0, _R)], bb, ld).wait()
        pltpu.make_async_copy(ib_hbm.at[pl.ds(0, _R)], ibuf, ld).wait()
        pltpu.make_async_copy(g_hbm.at[pl.ds(0, _R)], gb, ld).wait()
        pltpu.make_async_copy(ig_hbm.at[pl.ds(0, _R)], igbuf, ld).wait()

    def start_scatter(s):
        bb, gb, ibuf, igbuf, _, st = bufs[s]
        pltpu.async_copy(bb, accb_sh.at[ibuf], st, add=True)
        pltpu.async_copy(gb, accg_sh.at[igbuf], st, add=True)

    def wait_scatter(s):
        bb, gb, ibuf, igbuf, _, st = bufs[s]
        pltpu.make_async_copy(bb, accb_sh.at[ibuf], st).wait()
        pltpu.make_async_copy(gb, accg_sh.at[igbuf], st).wait()

    nch = jnp.where(wid < _CHUNKS_EXTRA, _CHUNKS_BASE + 1, _CHUNKS_BASE)

    start_load(0, 0)

    @pl.when(sid == 0)
    def _init():
        pltpu.sync_copy(z_acc_hbm, accb_sh)
        pltpu.sync_copy(z_acc_hbm, accg_sh)

    plsc.subcore_barrier()

    def outer(k, carry):
        for s in (0, 1):
            i = 2 * k + s

            @pl.when(i < nch)
            def _step():
                wait_load(s)
                start_scatter(s)

                @pl.when(i + 1 < nch)
                def _prefetch():
                    @pl.when(i >= 1)
                    def _drain():
                        wait_scatter(1 - s)

                    start_load(i + 1, 1 - s)

        return carry

    lax.fori_loop(0, (_CHUNKS_BASE + 2) // 2, outer, 0)
    wait_scatter(0)
    wait_scatter(1)
    plsc.subcore_barrier()

    @pl.when(sid == 0)
    def _flush():
        pltpu.sync_copy(accb_sh, accb_out.at[cid])
        pltpu.sync_copy(accg_sh, accg_out.at[cid])


def _hist16(ids_row):
    hi = lax.shift_right_logical(ids_row, 4)
    lo = ids_row & 15
    k = lax.broadcasted_iota(jnp.int32, (16, _BK), 0)
    oh_hi = (k == hi).astype(jnp.float32)
    oh_lo = (k == lo).astype(jnp.float32)
    return lax.dot_general(oh_hi, oh_lo, (((1,), (1,)), ((), ())),
                           preferred_element_type=jnp.float32)


def _expand_counts(c16):
    g = lax.broadcasted_iota(jnp.int32, (_S, 16), 0)
    k = lax.broadcasted_iota(jnp.int32, (_S, 16), 1)
    sel_hi = (g // 16 == k).astype(jnp.float32)
    sel_lo = (g % 16 == k).astype(jnp.float32)
    rows = jnp.dot(sel_hi, c16, preferred_element_type=jnp.float32)
    return jnp.sum(rows * sel_lo, axis=1, keepdims=True)


def _counts_kernel(idb_ref, idg_ref, cb_out, cg_out):
    cb16 = jnp.zeros((16, 16), jnp.float32)
    cg16 = jnp.zeros((16, 16), jnp.float32)
    for j in range(_NBLK):
        cb16 += _hist16(idb_ref[pl.ds(j * _BK, _BK)])
        cg16 += _hist16(idg_ref[pl.ds(j * _BK, _BK)])
    cb_out[:, :] = _expand_counts(cb16)
    cg_out[:, :] = _expand_counts(cg16)


def _tc_tail_segsum(idb_ref, idg_ref, bz_ref, gz_ref, tcb_out, tcg_out,
                    accb, accg):
    i = pl.program_id(0)

    @pl.when(i == 0)
    def _zero():
        accb[:, :] = jnp.zeros((_S, _C), jnp.float32)
        accg[:, :] = jnp.zeros((_S, _C), jnp.float32)

    g = lax.broadcasted_iota(jnp.int32, (_S, _TBK), 0)
    ohb = (g == idb_ref[0]).astype(jnp.float32)
    ohg = (g == idg_ref[0]).astype(jnp.float32)
    accb[:, :] += jnp.dot(ohb, bz_ref[:, :], preferred_element_type=jnp.float32)
    accg[:, :] += jnp.dot(ohg, gz_ref[:, :], preferred_element_type=jnp.float32)

    @pl.when(i == _NT - 1)
    def _final():
        tcb_out[:, :] = accb[:, :]
        tcg_out[:, :] = accg[:, :]


def _combine_kernel(accb_ref, accg_ref, tcb_ref, tcg_ref, cb_ref, cg_ref,
                    w_ref, bias_ref, out_ref):
    sb = accb_ref[0] + accb_ref[1] + tcb_ref[:, :]
    sg = accg_ref[0] + accg_ref[1] + tcg_ref[:, :]
    hb = sb / jnp.maximum(cb_ref[:, :], 1.0)
    hg = sg / jnp.maximum(cg_ref[:, :], 1.0)
    w1 = w_ref[0:_C, :]
    w2 = w_ref[_C:2 * _C, :]
    out_ref[:, :] = (
        jnp.dot(hb, w1, preferred_element_type=jnp.float32)
        + jnp.dot(hg, w2, preferred_element_type=jnp.float32)
        + bias_ref[0, 0]
    )


def kernel(B_z, G_z, x_b_batch, x_g_batch, W, b):
    ib = x_b_batch.astype(jnp.int32)
    ig = x_g_batch.astype(jnp.int32)
    z_acc = jnp.zeros((_S, _C), jnp.float32)

    mesh = plsc.VectorSubcoreMesh(core_axis_name="c", subcore_axis_name="s")
    sc = pl.kernel(
        _sc_segment_sums,
        out_type=(
            jax.ShapeDtypeStruct((2, _S, _C), jnp.float32),
            jax.ShapeDtypeStruct((2, _S, _C), jnp.float32),
        ),
        mesh=mesh,
        scratch_types=[
            pltpu.VMEM((_R, _C), jnp.float32),
            pltpu.VMEM((_R, _C), jnp.float32),
            pltpu.VMEM((_R, _C), jnp.float32),
            pltpu.VMEM((_R, _C), jnp.float32),
            pltpu.VMEM((_R,), jnp.int32),
            pltpu.VMEM((_R,), jnp.int32),
            pltpu.VMEM((_R,), jnp.int32),
            pltpu.VMEM((_R,), jnp.int32),
            pltpu.VMEM_SHARED((_S, _C), jnp.float32),
            pltpu.VMEM_SHARED((_S, _C), jnp.float32),
            pltpu.SemaphoreType.DMA,
            pltpu.SemaphoreType.DMA,
            pltpu.SemaphoreType.DMA,
            pltpu.SemaphoreType.DMA,
        ],
    )
    accb, accg = sc(B_z, G_z, ib, ig, z_acc)

    cb, cg = pl.pallas_call(
        _counts_kernel,
        grid=(1,),
        in_specs=[
            pl.BlockSpec((_NB,), lambda i: (0,)),
            pl.BlockSpec((_NB,), lambda i: (0,)),
        ],
        out_specs=[
            pl.BlockSpec((_S, 1), lambda i: (0, 0)),
            pl.BlockSpec((_S, 1), lambda i: (0, 0)),
        ],
        out_shape=[
            jax.ShapeDtypeStruct((_S, 1), jnp.float32),
            jax.ShapeDtypeStruct((_S, 1), jnp.float32),
        ],
    )(ib, ig)

    tcb, tcg = pl.pallas_call(
        _tc_tail_segsum,
        grid=(_NT,),
        in_specs=[
            pl.BlockSpec((1, 1, _TBK), lambda i: (i + _OFF, 0, 0)),
            pl.BlockSpec((1, 1, _TBK), lambda i: (i + _OFF, 0, 0)),
            pl.BlockSpec((_TBK, _C), lambda i: (i + _OFF, 0)),
            pl.BlockSpec((_TBK, _C), lambda i: (i + _OFF, 0)),
        ],
        out_specs=[
            pl.BlockSpec((_S, _C), lambda i: (0, 0)),
            pl.BlockSpec((_S, _C), lambda i: (0, 0)),
        ],
        out_shape=[
            jax.ShapeDtypeStruct((_S, _C), jnp.float32),
            jax.ShapeDtypeStruct((_S, _C), jnp.float32),
        ],
        scratch_shapes=[
            pltpu.VMEM((_S, _C), jnp.float32),
            pltpu.VMEM((_S, _C), jnp.float32),
        ],
    )(ib.reshape(_NB // _TBK, 1, _TBK), ig.reshape(_NB // _TBK, 1, _TBK),
      B_z, G_z)

    out = pl.pallas_call(
        _combine_kernel,
        out_shape=jax.ShapeDtypeStruct((_S, 1), jnp.float32),
    )(accb, accg, tcb, tcg, cb, cg, W, b.reshape(1, 1))
    return out


# speedup vs baseline: 13.7216x; 1.0450x over previous
import jax
import jax.numpy as jnp
from jax import lax
from jax.experimental import pallas as pl
from jax.experimental.pallas import tpu as pltpu
from jax.experimental.pallas import tpu_sc as plsc

_NB = 100000
_C = 128
_S = 256
_R = 112
_ROWS_SC = 56000
_NCH = _ROWS_SC // _R
_NW = 32
_CHUNKS_BASE = _NCH // _NW
_CHUNKS_EXTRA = _NCH % _NW

_BK = 25000
_NBLK = _NB // _BK

_TBK = 2000
_NT = (_NB - _ROWS_SC) // _TBK
_OFF = _ROWS_SC // _TBK


def _sc_segment_sums(b_hbm, g_hbm, ib_hbm, ig_hbm, z_acc_hbm,
                     accb_out, accg_out,
                     bb0, bb1, gb0, gb1, ib0, ib1, ig0, ig1,
                     accb_sh, accg_sh, ld0, ld1, st0, st1):
    cid = lax.axis_index("c")
    sid = lax.axis_index("s")
    wid = cid * 16 + sid

    bufs = ((bb0, gb0, ib0, ig0, ld0, st0), (bb1, gb1, ib1, ig1, ld1, st1))

    def start_load(i, s):
        bb, gb, ibuf, igbuf, ld, _ = bufs[s]
        base = (wid + i * _NW) * _R
        pltpu.async_copy(b_hbm.at[pl.ds(base, _R)], bb, ld)
        pltpu.async_copy(ib_hbm.at[pl.ds(base, _R)], ibuf, ld)
        pltpu.async_copy(g_hbm.at[pl.ds(base, _R)], gb, ld)
        pltpu.async_copy(ig_hbm.at[pl.ds(base, _R)], igbuf, ld)

    def wait_load(s):
        bb, gb, ibuf, igbuf, ld, _ = bufs[s]
        pltpu.make_async_copy(b_hbm.at[pl.ds(0, _R)], bb, ld).wait()
        pltpu.make_async_copy(ib_hbm.at[pl.ds(0, _R)], ibuf, ld).wait()
        pltpu.make_async_copy(g_hbm.at[pl.ds(0, _R)], gb, ld).wait()
        pltpu.make_async_copy(ig_hbm.at[pl.ds(0, _R)], igbuf, ld).wait()

    def start_scatter(s):
        bb, gb, ibuf, igbuf, _, st = bufs[s]
        pltpu.async_copy(bb, accb_sh.at[ibuf], st, add=True)
        pltpu.async_copy(gb, accg_sh.at[igbuf], st, add=True)

    def wait_scatter(s):
        bb, gb, ibuf, igbuf, _, st = bufs[s]
        pltpu.make_async_copy(bb, accb_sh.at[ibuf], st).wait()
        pltpu.make_async_copy(gb, accg_sh.at[igbuf], st).wait()

    nch = jnp.where(wid < _CHUNKS_EXTRA, _CHUNKS_BASE + 1, _CHUNKS_BASE)

    start_load(0, 0)

    @pl.when(sid == 0)
    def _init():
        pltpu.sync_copy(z_acc_hbm, accb_sh)
        pltpu.sync_copy(z_acc_hbm, accg_sh)

    plsc.subcore_barrier()

    def outer(k, carry):
        for s in (0, 1):
            i = 2 * k + s

            @pl.when(i < nch)
            def _step():
                wait_load(s)
                start_scatter(s)

                @pl.when(i + 1 < nch)
                def _prefetch():
                    @pl.when(i >= 1)
                    def _drain():
                        wait_scatter(1 - s)

                    start_load(i + 1, 1 - s)

        return carry

    lax.fori_loop(0, (_CHUNKS_BASE + 2) // 2, outer, 0)
    wait_scatter(0)
    wait_scatter(1)
    plsc.subcore_barrier()

    @pl.when(sid == 0)
    def _flush():
        pltpu.sync_copy(accb_sh, accb_out.at[cid])
        pltpu.sync_copy(accg_sh, accg_out.at[cid])


def _hist16(ids_row):
    hi = lax.shift_right_logical(ids_row, 4)
    lo = ids_row & 15
    k = lax.broadcasted_iota(jnp.int32, (16, _BK), 0)
    oh_hi = (k == hi).astype(jnp.float32)
    oh_lo = (k == lo).astype(jnp.float32)
    return lax.dot_general(oh_hi, oh_lo, (((1,), (1,)), ((), ())),
                           preferred_element_type=jnp.float32)


def _expand_counts(c16):
    g = lax.broadcasted_iota(jnp.int32, (_S, 16), 0)
    k = lax.broadcasted_iota(jnp.int32, (_S, 16), 1)
    sel_hi = (g // 16 == k).astype(jnp.float32)
    sel_lo = (g % 16 == k).astype(jnp.float32)
    rows = jnp.dot(sel_hi, c16, preferred_element_type=jnp.float32)
    return jnp.sum(rows * sel_lo, axis=1, keepdims=True)


def _counts_kernel(idb_ref, idg_ref, cb_out, cg_out):
    cb16 = jnp.zeros((16, 16), jnp.float32)
    cg16 = jnp.zeros((16, 16), jnp.float32)
    for j in range(_NBLK):
        cb16 += _hist16(idb_ref[pl.ds(j * _BK, _BK)])
        cg16 += _hist16(idg_ref[pl.ds(j * _BK, _BK)])
    cb_out[:, :] = _expand_counts(cb16)
    cg_out[:, :] = _expand_counts(cg16)


def _tc_tail_segsum(idb_ref, idg_ref, bz_ref, gz_ref, tcb_out, tcg_out,
                    accb, accg):
    i = pl.program_id(0)

    @pl.when(i == 0)
    def _zero():
        accb[:, :] = jnp.zeros((_S, _C), jnp.float32)
        accg[:, :] = jnp.zeros((_S, _C), jnp.float32)

    g = lax.broadcasted_iota(jnp.int32, (_S, _TBK), 0)
    ohb = (g == idb_ref[0]).astype(jnp.float32)
    ohg = (g == idg_ref[0]).astype(jnp.float32)
    accb[:, :] += jnp.dot(ohb, bz_ref[:, :], preferred_element_type=jnp.float32)
    accg[:, :] += jnp.dot(ohg, gz_ref[:, :], preferred_element_type=jnp.float32)

    @pl.when(i == _NT - 1)
    def _final():
        tcb_out[:, :] = accb[:, :]
        tcg_out[:, :] = accg[:, :]


def _combine_kernel(accb_ref, accg_ref, tcb_ref, tcg_ref, cb_ref, cg_ref,
                    w_ref, bias_ref, out_ref):
    sb = accb_ref[0] + accb_ref[1] + tcb_ref[:, :]
    sg = accg_ref[0] + accg_ref[1] + tcg_ref[:, :]
    hb = sb / jnp.maximum(cb_ref[:, :], 1.0)
    hg = sg / jnp.maximum(cg_ref[:, :], 1.0)
    w1 = w_ref[0:_C, :]
    w2 = w_ref[_C:2 * _C, :]
    out_ref[:, :] = (
        jnp.dot(hb, w1, preferred_element_type=jnp.float32)
        + jnp.dot(hg, w2, preferred_element_type=jnp.float32)
        + bias_ref[0, 0]
    )


def kernel(B_z, G_z, x_b_batch, x_g_batch, W, b):
    ib = x_b_batch.astype(jnp.int32)
    ig = x_g_batch.astype(jnp.int32)
    z_acc = jnp.zeros((_S, _C), jnp.float32)

    mesh = plsc.VectorSubcoreMesh(core_axis_name="c", subcore_axis_name="s")
    sc = pl.kernel(
        _sc_segment_sums,
        out_type=(
            jax.ShapeDtypeStruct((2, _S, _C), jnp.float32),
            jax.ShapeDtypeStruct((2, _S, _C), jnp.float32),
        ),
        mesh=mesh,
        scratch_types=[
            pltpu.VMEM((_R, _C), jnp.float32),
            pltpu.VMEM((_R, _C), jnp.float32),
            pltpu.VMEM((_R, _C), jnp.float32),
            pltpu.VMEM((_R, _C), jnp.float32),
            pltpu.VMEM((_R,), jnp.int32),
            pltpu.VMEM((_R,), jnp.int32),
            pltpu.VMEM((_R,), jnp.int32),
            pltpu.VMEM((_R,), jnp.int32),
            pltpu.VMEM_SHARED((_S, _C), jnp.float32),
            pltpu.VMEM_SHARED((_S, _C), jnp.float32),
            pltpu.SemaphoreType.DMA,
            pltpu.SemaphoreType.DMA,
            pltpu.SemaphoreType.DMA,
            pltpu.SemaphoreType.DMA,
        ],
    )
    accb, accg = sc(B_z, G_z, ib, ig, z_acc)

    cb, cg = pl.pallas_call(
        _counts_kernel,
        grid=(1,),
        in_specs=[
            pl.BlockSpec((_NB,), lambda i: (0,)),
            pl.BlockSpec((_NB,), lambda i: (0,)),
        ],
        out_specs=[
            pl.BlockSpec((_S, 1), lambda i: (0, 0)),
            pl.BlockSpec((_S, 1), lambda i: (0, 0)),
        ],
        out_shape=[
            jax.ShapeDtypeStruct((_S, 1), jnp.float32),
            jax.ShapeDtypeStruct((_S, 1), jnp.float32),
        ],
    )(ib, ig)

    tcb, tcg = pl.pallas_call(
        _tc_tail_segsum,
        grid=(_NT,),
        in_specs=[
            pl.BlockSpec((1, 1, _TBK), lambda i: (i + _OFF, 0, 0)),
            pl.BlockSpec((1, 1, _TBK), lambda i: (i + _OFF, 0, 0)),
            pl.BlockSpec((_TBK, _C), lambda i: (i + _OFF, 0)),
            pl.BlockSpec((_TBK, _C), lambda i: (i + _OFF, 0)),
        ],
        out_specs=[
            pl.BlockSpec((_S, _C), lambda i: (0, 0)),
            pl.BlockSpec((_S, _C), lambda i: (0, 0)),
        ],
        out_shape=[
            jax.ShapeDtypeStruct((_S, _C), jnp.float32),
            jax.ShapeDtypeStruct((_S, _C), jnp.float32),
        ],
        scratch_shapes=[
            pltpu.VMEM((_S, _C), jnp.float32),
            pltpu.VMEM((_S, _C), jnp.float32),
        ],
    )(ib.reshape(_NB // _TBK, 1, _TBK), ig.reshape(_NB // _TBK, 1, _TBK),
      B_z, G_z)

    out = pl.pallas_call(
        _combine_kernel,
        out_shape=jax.ShapeDtypeStruct((_S, 1), jnp.float32),
    )(accb, accg, tcb, tcg, cb, cg, W, b.reshape(1, 1))
    return out
